# baseline (speedup 1.0000x reference)
import jax
import jax.numpy as jnp
from jax import lax
from jax.experimental import pallas as pl
from jax.experimental.pallas import tpu as pltpu
from jax.experimental.pallas import tpu_sc as plsc

D_VOCAB = 100000
D_MODEL = 768
BATCH = 4
SEQ = 4096

NC = 2
NS = 16
NW = NC * NS

TOKENS_TOTAL = BATCH * SEQ
PER_W = TOKENS_TOTAL // NW
CHUNK = 64
CHUNKS = PER_W // CHUNK
NBUF = 2


W_PER_ROW = SEQ // PER_W


def _embed_sc(tokens, W_E):
    mesh = plsc.VectorSubcoreMesh(core_axis_name="c", subcore_axis_name="s")

    @pl.kernel(
        mesh=mesh,
        out_type=jax.ShapeDtypeStruct((TOKENS_TOTAL, D_MODEL), jnp.float32),
        scratch_types=(
            [pltpu.VMEM((PER_W,), jnp.int32)]
            + [pltpu.VMEM((CHUNK, D_MODEL), jnp.float32)] * NBUF
            + [pltpu.SemaphoreType.DMA] * (2 * NBUF)
        ),
    )
    def k(tok_hbm, table_hbm, out_hbm, idx_v, *bufs):
        rows = list(bufs[:NBUF])
        gsem = list(bufs[NBUF:2 * NBUF])
        ssem = list(bufs[2 * NBUF:])
        wid = lax.axis_index("s") * NC + lax.axis_index("c")
        base = wid * PER_W
        pltpu.sync_copy(
            tok_hbm.at[wid // W_PER_ROW, pl.ds((wid % W_PER_ROW) * PER_W, PER_W)],
            idx_v)
        LOOK = NBUF - 2
        gh, sh = [None] * NBUF, [None] * NBUF
        for b in range(LOOK):
            gh[b] = pltpu.async_copy(table_hbm.at[idx_v.at[pl.ds(b * CHUNK, CHUNK)]], rows[b], gsem[b])
        for j in range(CHUNKS):
            b = j % NBUF
            m = j + LOOK
            if m < CHUNKS:
                bm = m % NBUF
                if sh[bm] is not None:
                    sh[bm].wait()
                gh[bm] = pltpu.async_copy(
                    table_hbm.at[idx_v.at[pl.ds(m * CHUNK, CHUNK)]], rows[bm], gsem[bm])
            gh[b].wait()
            sh[b] = pltpu.async_copy(
                rows[b], out_hbm.at[pl.ds(base + j * CHUNK, CHUNK)], ssem[b])
        for b in range(NBUF):
            if sh[b] is not None:
                sh[b].wait()

    return k(tokens, W_E)


def kernel(tokens, W_E):
    emb = _embed_sc(tokens, W_E)
    return (tokens, emb.reshape(BATCH, SEQ, D_MODEL))


# speedup vs baseline: 1.0616x; 1.0616x over previous
import jax
import jax.numpy as jnp
from jax import lax
from jax.experimental import pallas as pl
from jax.experimental.pallas import tpu as pltpu
from jax.experimental.pallas import tpu_sc as plsc

D_VOCAB = 100000
D_MODEL = 768
BATCH = 4
SEQ = 4096

NC = 2
NS = 16
NW = NC * NS

TOKENS_TOTAL = BATCH * SEQ
PER_W = TOKENS_TOTAL // NW
CHUNK = 32
CHUNKS = PER_W // CHUNK
NBUF = 5


W_PER_ROW = SEQ // PER_W


def _embed_sc(tokens, W_E):
    mesh = plsc.VectorSubcoreMesh(core_axis_name="c", subcore_axis_name="s")

    @pl.kernel(
        mesh=mesh,
        out_type=jax.ShapeDtypeStruct((TOKENS_TOTAL, D_MODEL), jnp.float32),
        scratch_types=(
            [pltpu.VMEM((PER_W,), jnp.int32)]
            + [pltpu.VMEM((CHUNK, D_MODEL), jnp.float32)] * NBUF
            + [pltpu.SemaphoreType.DMA] * (2 * NBUF)
        ),
    )
    def k(tok_hbm, table_hbm, out_hbm, idx_v, *bufs):
        rows = list(bufs[:NBUF])
        gsem = list(bufs[NBUF:2 * NBUF])
        ssem = list(bufs[2 * NBUF:])
        wid = lax.axis_index("s") * NC + lax.axis_index("c")
        base = wid * PER_W
        pltpu.sync_copy(
            tok_hbm.at[wid // W_PER_ROW, pl.ds((wid % W_PER_ROW) * PER_W, PER_W)],
            idx_v)
        LOOK = NBUF - 2
        gh, sh = [None] * NBUF, [None] * NBUF
        for b in range(LOOK):
            gh[b] = pltpu.async_copy(table_hbm.at[idx_v.at[pl.ds(b * CHUNK, CHUNK)]], rows[b], gsem[b])
        for j in range(CHUNKS):
            b = j % NBUF
            m = j + LOOK
            if m < CHUNKS:
                bm = m % NBUF
                if sh[bm] is not None:
                    sh[bm].wait()
                gh[bm] = pltpu.async_copy(
                    table_hbm.at[idx_v.at[pl.ds(m * CHUNK, CHUNK)]], rows[bm], gsem[bm])
            gh[b].wait()
            sh[b] = pltpu.async_copy(
                rows[b], out_hbm.at[pl.ds(base + j * CHUNK, CHUNK)], ssem[b])
        for b in range(NBUF):
            if sh[b] is not None:
                sh[b].wait()

    return k(tokens, W_E)


def kernel(tokens, W_E):
    emb = _embed_sc(tokens, W_E)
    return (tokens, emb.reshape(BATCH, SEQ, D_MODEL))


# speedup vs baseline: 1.0697x; 1.0076x over previous
import jax
import jax.numpy as jnp
from jax import lax
from jax.experimental import pallas as pl
from jax.experimental.pallas import tpu as pltpu
from jax.experimental.pallas import tpu_sc as plsc

D_VOCAB = 100000
D_MODEL = 768
BATCH = 4
SEQ = 4096

NC = 2
NS = 16
NW = NC * NS

TOKENS_TOTAL = BATCH * SEQ
PER_W = TOKENS_TOTAL // NW
CHUNK = 32
CHUNKS = PER_W // CHUNK
NBUF = 5


W_PER_ROW = SEQ // PER_W


def _embed_sc(tokens, W_E):
    mesh = plsc.VectorSubcoreMesh(core_axis_name="c", subcore_axis_name="s")

    @pl.kernel(
        mesh=mesh,
        out_type=(
            jax.ShapeDtypeStruct((TOKENS_TOTAL, D_MODEL), jnp.float32),
            jax.ShapeDtypeStruct((BATCH, SEQ), jnp.int32),
        ),
        scratch_types=(
            [pltpu.VMEM((PER_W,), jnp.int32)]
            + [pltpu.VMEM((CHUNK, D_MODEL), jnp.float32)] * NBUF
            + [pltpu.SemaphoreType.DMA] * (2 * NBUF)
        ),
    )
    def k(tok_hbm, table_hbm, out_hbm, tok_out_hbm, idx_v, *bufs):
        rows = list(bufs[:NBUF])
        gsem = list(bufs[NBUF:2 * NBUF])
        ssem = list(bufs[2 * NBUF:])
        wid = lax.axis_index("s") * NC + lax.axis_index("c")
        base = wid * PER_W
        pltpu.sync_copy(
            tok_hbm.at[wid // W_PER_ROW, pl.ds((wid % W_PER_ROW) * PER_W, PER_W)],
            idx_v)
        pltpu.sync_copy(
            idx_v,
            tok_out_hbm.at[wid // W_PER_ROW,
                           pl.ds((wid % W_PER_ROW) * PER_W, PER_W)])
        LOOK = NBUF - 2
        gh, sh = [None] * NBUF, [None] * NBUF
        for b in range(LOOK):
            gh[b] = pltpu.async_copy(table_hbm.at[idx_v.at[pl.ds(b * CHUNK, CHUNK)]], rows[b], gsem[b])
        for j in range(CHUNKS):
            b = j % NBUF
            m = j + LOOK
            if m < CHUNKS:
                bm = m % NBUF
                if sh[bm] is not None:
                    sh[bm].wait()
                gh[bm] = pltpu.async_copy(
                    table_hbm.at[idx_v.at[pl.ds(m * CHUNK, CHUNK)]], rows[bm], gsem[bm])
            gh[b].wait()
            sh[b] = pltpu.async_copy(
                rows[b], out_hbm.at[pl.ds(base + j * CHUNK, CHUNK)], ssem[b])
        for b in range(NBUF):
            if sh[b] is not None:
                sh[b].wait()

    return k(tokens, W_E)


def kernel(tokens, W_E):
    emb, tok_out = _embed_sc(tokens, W_E)
    return (tok_out, emb.reshape(BATCH, SEQ, D_MODEL))
